# Initial kernel scaffold; baseline (speedup 1.0000x reference)
import jax
import jax.numpy as jnp
from jax.experimental import pallas as pl


def kernel(x01, tables):
    raise NotImplementedError("write your pallas kernel here")


# speedup vs baseline: 34.9179x; 34.9179x over previous
import math

import jax
import jax.numpy as jnp
from jax import lax
from jax.experimental import pallas as pl
from jax.experimental.pallas import tpu as pltpu
from jax.experimental.pallas import tpu_sc as plsc

_NUM_LEVELS = 16
_FEATS = 2
_TABLE = 2 ** 19
_MASK = _TABLE - 1
_MIN_RES = 16
_MAX_RES = 512
_P1 = 1540863
_P2 = 1256879
_P3 = 1957123

_GROWTH = math.exp(math.log(_MAX_RES / _MIN_RES) / (_NUM_LEVELS - 1))
_RES = [int(math.floor(_MIN_RES * _GROWTH ** l + 1e-06)) for l in range(_NUM_LEVELS)]

_CORNERS = [(ox, oy, oz) for ox in (0, 1) for oy in (0, 1) for oz in (0, 1)]

_NC = 2
_NS = 16
_NW = _NC * _NS

_C = 128
_G = _C // 16
_M = _NUM_LEVELS * 8 * _C
_IDXN = 2 * _M


def _hash_grid_sc(x2d, tab_flat, n_points):
    per_w = n_points // _NW
    n_chunks = per_w // _C
    out_rows = _NUM_LEVELS * _FEATS

    mesh = plsc.VectorSubcoreMesh(core_axis_name="c", subcore_axis_name="s")

    def body(x_hbm, tab_hbm, out_hbm, x_v, idx_v, w_v, rows_v, outb_v, sem):
        wid = lax.axis_index("s") * _NC + lax.axis_index("c")
        wbase = wid * per_w

        def chunk_body(c, carry):
            pbase = wbase + c * _C

            pltpu.sync_copy(x_hbm.at[:, pl.ds(pbase, _C)], x_v)

            def index_group(g, _):
                xv = jnp.clip(x_v[0, pl.ds(g * 16, 16)], 0.0, 1.0)
                yv = jnp.clip(x_v[1, pl.ds(g * 16, 16)], 0.0, 1.0)
                zv = jnp.clip(x_v[2, pl.ds(g * 16, 16)], 0.0, 1.0)
                for l in range(_NUM_LEVELS):
                    resf = float(_RES[l])
                    px = xv * resf
                    py = yv * resf
                    pz = zv * resf
                    ix0 = px.astype(jnp.int32)
                    iy0 = py.astype(jnp.int32)
                    iz0 = pz.astype(jnp.int32)
                    fx = px - ix0.astype(jnp.float32)
                    fy = py - iy0.astype(jnp.float32)
                    fz = pz - iz0.astype(jnp.float32)
                    hx = (ix0 * _P1, ix0 * _P1 + _P1)
                    hy = (iy0 * _P2, iy0 * _P2 + _P2)
                    hz = (iz0 * _P3, iz0 * _P3 + _P3)
                    wx = (1.0 - fx, fx)
                    wy = (1.0 - fy, fy)
                    wz = (1.0 - fz, fz)
                    lbase2 = l * _TABLE * 2
                    for j, (ox, oy, oz) in enumerate(_CORNERS):
                        h = (hx[ox] ^ hy[oy]) ^ hz[oz]
                        e0 = ((h & _MASK) << 1) + lbase2
                        wj = (wx[ox] * wy[oy]) * wz[oz]
                        off = (l * 8 + j) * _C + g * 16
                        idx_v[pl.ds(off, 16)] = e0
                        idx_v[pl.ds(off + _M, 16)] = e0 + 1
                        w_v[pl.ds(off, 16)] = wj
                return 0

            lax.fori_loop(0, _G, index_group, 0)

            pltpu.async_copy(tab_hbm.at[idx_v], rows_v, sem).wait()

            def acc_group(g, _):
                for l in range(_NUM_LEVELS):
                    acc0 = None
                    acc1 = None
                    for j in range(8):
                        off = (l * 8 + j) * _C + g * 16
                        wj = w_v[pl.ds(off, 16)]
                        r0 = rows_v[pl.ds(off, 16)]
                        r1 = rows_v[pl.ds(off + _M, 16)]
                        if acc0 is None:
                            acc0 = wj * r0
                            acc1 = wj * r1
                        else:
                            acc0 = acc0 + wj * r0
                            acc1 = acc1 + wj * r1
                    outb_v[2 * l, pl.ds(g * 16, 16)] = acc0
                    outb_v[2 * l + 1, pl.ds(g * 16, 16)] = acc1
                return 0

            lax.fori_loop(0, _G, acc_group, 0)

            pltpu.sync_copy(outb_v, out_hbm.at[:, pl.ds(pbase, _C)])
            return carry

        lax.fori_loop(0, n_chunks, chunk_body, 0)

    kern = pl.kernel(
        body,
        out_type=jax.ShapeDtypeStruct((out_rows, n_points), jnp.float32),
        mesh=mesh,
        scratch_types=[
            pltpu.VMEM((3, _C), jnp.float32),
            pltpu.VMEM((_IDXN,), jnp.int32),
            pltpu.VMEM((_M,), jnp.float32),
            pltpu.VMEM((_IDXN,), jnp.float32),
            pltpu.VMEM((_NUM_LEVELS * _FEATS, _C), jnp.float32),
            pltpu.SemaphoreType.DMA,
        ],
    )
    return kern(x2d, tab_flat)


def kernel(x01, tables):
    n = x01.shape[0]
    x2d = x01.T
    tab_flat = tables.reshape(-1)
    out_t = _hash_grid_sc(x2d, tab_flat, n)
    return out_t.T
